# baseline (speedup 1.0000x reference)
import jax
import jax.numpy as jnp
from jax.experimental import pallas as pl
from jax.experimental.pallas import tpu as pltpu

_N_PWL = 16
_K = 10
_L = _K * _K
_M = 100
_BBLK = 512


def _pwl_tables(fw):
    n1 = _N_PWL + 1
    w = jnp.abs(fw)
    w = w / jnp.sum(w, axis=1, keepdims=True)
    tri = (
        jax.lax.broadcasted_iota(jnp.int32, (n1, n1), 0)
        <= jax.lax.broadcasted_iota(jnp.int32, (n1, n1), 1)
    ).astype(jnp.float32)
    csum = jax.lax.dot_general(
        w, tri, (((1,), (0,)), ((), ())),
        precision=jax.lax.Precision.HIGHEST,
        preferred_element_type=jnp.float32,
    )
    return w, csum


def _pwl_multi(tabs, fids, x):
    n = _N_PWL
    y = n * x
    idx = y.astype(jnp.int32)
    fr = y - idx.astype(y.dtype)
    ci = jnp.clip(idx, 0, n)
    ci2 = jnp.clip(idx + 1, 0, n)
    outs = []
    for i in fids:
        cs_t, w_t = tabs[i]
        c = jnp.take_along_axis(cs_t, ci, axis=1)
        ww = jnp.take_along_axis(w_t, ci2, axis=1)
        outs.append(c + fr * ww)
    return outs


def _counter_kernel(boxes_ref, att_ref, fw_ref, out_ref):
    f32 = jnp.float32
    S = _BBLK
    w, csum = _pwl_tables(fw_ref[...])
    tabs = {
        i: (jnp.broadcast_to(csum[i:i + 1, :], (S, _N_PWL + 1)),
            jnp.broadcast_to(w[i:i + 1, :], (S, _N_PWL + 1)))
        for i in (0, 1, 2, 3, 4, 5, 6, 7)
    }

    dot = lambda a, b: jax.lax.dot_general(
        a, b, (((1,), (0,)), ((), ())),
        precision=jax.lax.Precision.HIGHEST,
        preferred_element_type=f32,
    )

    att_full = att_ref[...]
    att_cur = att_full
    lane = jax.lax.broadcasted_iota(jnp.int32, (S, _M), 1)
    col10 = jax.lax.broadcasted_iota(jnp.int32, (S, _K), 1)
    idx10 = jnp.zeros((S, _K), jnp.int32)
    for j in range(_K):
        m = jnp.max(att_cur, axis=1, keepdims=True)
        ismax = att_cur == m
        first = jnp.min(jnp.where(ismax, lane, _M), axis=1, keepdims=True)
        idx10 = idx10 + first * (col10 == j).astype(jnp.int32)
        att_cur = jnp.where(lane == first, -jnp.inf, att_cur)

    att_top = jnp.take_along_axis(att_full, idx10, axis=1)
    bf = [jnp.take_along_axis(boxes_ref[:, c, :], idx10, axis=1)
          for c in range(4)]
    att = jax.nn.sigmoid(att_top)

    r10 = jax.lax.broadcasted_iota(jnp.int32, (_K, _L), 0)
    l100 = jax.lax.broadcasted_iota(jnp.int32, (_K, _L), 1)
    E1 = (l100 // _K == r10).astype(f32)
    E2 = (l100 % _K == r10).astype(f32)
    r100 = jax.lax.broadcasted_iota(jnp.int32, (_L, _K), 0)
    c10 = jax.lax.broadcasted_iota(jnp.int32, (_L, _K), 1)
    R = (r100 // _K == c10).astype(f32)
    e1 = lambda v: dot(v, E1)
    idxE2 = jax.lax.broadcasted_iota(jnp.int32, (S, _L), 1) % _K
    e2 = lambda v: jnp.take_along_axis(v, idxE2, axis=1)

    atti, attj = e1(att), e2(att)
    relevancy = atti * attj

    x1, y1, x2, y2 = bf
    ix = jnp.maximum(
        jnp.minimum(e1(x2), e2(x2)) - jnp.maximum(e1(x1), e2(x1)), 0.0)
    iy = jnp.maximum(
        jnp.minimum(e1(y2), e2(y2)) - jnp.maximum(e1(y1), e2(y1)), 0.0)
    inter = ix * iy
    area = jnp.maximum(x2 - x1, 0.0) * jnp.maximum(y2 - y1, 0.0)
    iou = inter / (e1(area) + e2(area) - inter + 1e-12)
    distance = 1.0 - iou

    f0_rel, f3_rel = _pwl_multi(tabs, [0, 3], relevancy)
    f1_dist, f4_dist, f6_dist = _pwl_multi(tabs, [1, 4, 6], distance)
    score = f0_rel * f1_dist
    dedup = f3_rel * f4_dist

    att_diff = jnp.abs(atti - attj)
    terms = [_pwl_multi(tabs, [2], 1.0 - att_diff)[0]]
    for a in range(_K):
        da = dedup[:, a * _K:(a + 1) * _K]
        terms.append(
            _pwl_multi(tabs, [2], 1.0 - jnp.abs(e1(da) - e2(da)))[0])
    while len(terms) > 1:
        terms = [terms[i] * terms[i + 1] for i in range(0, len(terms) - 1, 2)] \
            + ([terms[-1]] if len(terms) % 2 else [])
    sim = terms[0]

    row_sims = dot(sim, R)
    all_sims = e1(row_sims) * e2(row_sims)
    score = score / all_sims
    correction = _pwl_multi(tabs, [0], att * att)[0] / row_sims
    total = (jnp.sum(score, axis=1, keepdims=True)
             + jnp.sum(correction, axis=1, keepdims=True))
    sc = jnp.sqrt(total + 1e-20)

    s_ = jnp.clip(sc, 0.0, float(_K))
    i_ = s_.astype(jnp.int32)
    fr_ = s_ - i_.astype(f32)
    io11 = jax.lax.broadcasted_iota(jnp.int32, (S, _K + 1), 1)
    tl = (io11 == jnp.clip(i_, 0, _K)).astype(f32)
    tr = (io11 == jnp.clip(i_ + 1, 0, _K)).astype(f32)
    one_hot = (1.0 - fr_) * tl + fr_ * tr

    att_conf = jnp.abs(_pwl_multi(tabs, [5], att)[0] - 0.5)
    dist_conf = jnp.abs(f6_dist - 0.5)
    mean_conf = (jnp.sum(att_conf, axis=1, keepdims=True) / _K
                 + jnp.sum(dist_conf, axis=1, keepdims=True) / _L)
    conf = _pwl_multi(tabs, [7], mean_conf)[0]

    out_ref[...] = one_hot * conf


def kernel(boxes, attention, f_weights):
    B = attention.shape[0]
    grid = B // _BBLK
    return pl.pallas_call(
        _counter_kernel,
        grid=(grid,),
        in_specs=[
            pl.BlockSpec((_BBLK, 4, _M), lambda i: (i, 0, 0)),
            pl.BlockSpec((_BBLK, _M), lambda i: (i, 0)),
            pl.BlockSpec((16, _N_PWL + 1), lambda i: (0, 0)),
        ],
        out_specs=pl.BlockSpec((_BBLK, _K + 1), lambda i: (i, 0)),
        out_shape=jax.ShapeDtypeStruct((B, _K + 1), jnp.float32),
        compiler_params=pltpu.CompilerParams(
            dimension_semantics=("parallel",)),
    )(boxes, attention, f_weights)


# speedup vs baseline: 1.2361x; 1.2361x over previous
import jax
import jax.numpy as jnp
from jax.experimental import pallas as pl
from jax.experimental.pallas import tpu as pltpu

_N_PWL = 16
_K = 10
_L = _K * _K
_M = 100
_BBLK = 512


def _pwl_tables(fw):
    n1 = _N_PWL + 1
    w = jnp.abs(fw)
    w = w / jnp.sum(w, axis=1, keepdims=True)
    tri = (
        jax.lax.broadcasted_iota(jnp.int32, (n1, n1), 0)
        <= jax.lax.broadcasted_iota(jnp.int32, (n1, n1), 1)
    ).astype(jnp.float32)
    csum = jax.lax.dot_general(
        w, tri, (((1,), (0,)), ((), ())),
        precision=jax.lax.Precision.HIGHEST,
        preferred_element_type=jnp.float32,
    )
    return w, csum


def _pwl_multi(tabs, fids, x):
    n = _N_PWL
    y = n * x
    idx = y.astype(jnp.int32)
    fr = y - idx.astype(y.dtype)
    ci = jnp.clip(idx, 0, n)
    ci2 = jnp.clip(idx + 1, 0, n)
    outs = []
    for i in fids:
        cs_t, w_t = tabs[i]
        c = jnp.take_along_axis(cs_t, ci, axis=1)
        ww = jnp.take_along_axis(w_t, ci2, axis=1)
        outs.append(c + fr * ww)
    return outs


def _counter_kernel(boxes_ref, att_ref, fw_ref, out_ref):
    f32 = jnp.float32
    S = _BBLK
    w, csum = _pwl_tables(fw_ref[...])
    tabs = {
        i: (jnp.broadcast_to(csum[i:i + 1, :], (S, _N_PWL + 1)),
            jnp.broadcast_to(w[i:i + 1, :], (S, _N_PWL + 1)))
        for i in (0, 1, 2, 3, 4, 5, 6, 7)
    }

    dot = lambda a, b: jax.lax.dot_general(
        a, b, (((1,), (0,)), ((), ())),
        precision=jax.lax.Precision.HIGHEST,
        preferred_element_type=f32,
    )

    att_full = att_ref[...]
    att_cur = att_full
    lane = jax.lax.broadcasted_iota(jnp.int32, (S, _M), 1)
    col10 = jax.lax.broadcasted_iota(jnp.int32, (S, _K), 1)
    idx10 = jnp.zeros((S, _K), jnp.int32)
    for j in range(_K):
        m = jnp.max(att_cur, axis=1, keepdims=True)
        ismax = att_cur == m
        first = jnp.min(jnp.where(ismax, lane, _M), axis=1, keepdims=True)
        idx10 = idx10 + first * (col10 == j).astype(jnp.int32)
        att_cur = jnp.where(lane == first, -jnp.inf, att_cur)

    att_top = jnp.take_along_axis(att_full, idx10, axis=1)
    bf = [jnp.take_along_axis(boxes_ref[:, c, :], idx10, axis=1)
          for c in range(4)]
    att = jax.nn.sigmoid(att_top)

    r10 = jax.lax.broadcasted_iota(jnp.int32, (_K, _L), 0)
    l100 = jax.lax.broadcasted_iota(jnp.int32, (_K, _L), 1)
    E1 = (l100 // _K == r10).astype(f32)
    E2 = (l100 % _K == r10).astype(f32)
    r100 = jax.lax.broadcasted_iota(jnp.int32, (_L, _K), 0)
    c10 = jax.lax.broadcasted_iota(jnp.int32, (_L, _K), 1)
    R = (r100 // _K == c10).astype(f32)
    e1 = lambda v: dot(v, E1)
    e2 = lambda v: dot(v, E2)

    atti, attj = e1(att), e2(att)
    relevancy = atti * attj

    x1, y1, x2, y2 = bf
    ix = jnp.maximum(
        jnp.minimum(e1(x2), e2(x2)) - jnp.maximum(e1(x1), e2(x1)), 0.0)
    iy = jnp.maximum(
        jnp.minimum(e1(y2), e2(y2)) - jnp.maximum(e1(y1), e2(y1)), 0.0)
    inter = ix * iy
    area = jnp.maximum(x2 - x1, 0.0) * jnp.maximum(y2 - y1, 0.0)
    iou = inter / (e1(area) + e2(area) - inter + 1e-12)
    distance = 1.0 - iou

    f0_rel, f3_rel = _pwl_multi(tabs, [0, 3], relevancy)
    f1_dist, f4_dist, f6_dist = _pwl_multi(tabs, [1, 4, 6], distance)
    score = f0_rel * f1_dist
    dedup = f3_rel * f4_dist

    att_diff = jnp.abs(atti - attj)
    terms = [_pwl_multi(tabs, [2], 1.0 - att_diff)[0]]
    for a in range(_K):
        da = dedup[:, a * _K:(a + 1) * _K]
        terms.append(
            _pwl_multi(tabs, [2], 1.0 - jnp.abs(e1(da) - e2(da)))[0])
    while len(terms) > 1:
        terms = [terms[i] * terms[i + 1] for i in range(0, len(terms) - 1, 2)] \
            + ([terms[-1]] if len(terms) % 2 else [])
    sim = terms[0]

    row_sims = dot(sim, R)
    all_sims = e1(row_sims) * e2(row_sims)
    score = score / all_sims
    correction = _pwl_multi(tabs, [0], att * att)[0] / row_sims
    total = (jnp.sum(score, axis=1, keepdims=True)
             + jnp.sum(correction, axis=1, keepdims=True))
    sc = jnp.sqrt(total + 1e-20)

    s_ = jnp.clip(sc, 0.0, float(_K))
    i_ = s_.astype(jnp.int32)
    fr_ = s_ - i_.astype(f32)
    io11 = jax.lax.broadcasted_iota(jnp.int32, (S, _K + 1), 1)
    tl = (io11 == jnp.clip(i_, 0, _K)).astype(f32)
    tr = (io11 == jnp.clip(i_ + 1, 0, _K)).astype(f32)
    one_hot = (1.0 - fr_) * tl + fr_ * tr

    att_conf = jnp.abs(_pwl_multi(tabs, [5], att)[0] - 0.5)
    dist_conf = jnp.abs(f6_dist - 0.5)
    mean_conf = (jnp.sum(att_conf, axis=1, keepdims=True) / _K
                 + jnp.sum(dist_conf, axis=1, keepdims=True) / _L)
    conf = _pwl_multi(tabs, [7], mean_conf)[0]

    out_ref[...] = one_hot * conf


def kernel(boxes, attention, f_weights):
    B = attention.shape[0]
    grid = B // _BBLK
    return pl.pallas_call(
        _counter_kernel,
        grid=(grid,),
        in_specs=[
            pl.BlockSpec((_BBLK, 4, _M), lambda i: (i, 0, 0)),
            pl.BlockSpec((_BBLK, _M), lambda i: (i, 0)),
            pl.BlockSpec((16, _N_PWL + 1), lambda i: (0, 0)),
        ],
        out_specs=pl.BlockSpec((_BBLK, _K + 1), lambda i: (i, 0)),
        out_shape=jax.ShapeDtypeStruct((B, _K + 1), jnp.float32),
        compiler_params=pltpu.CompilerParams(
            dimension_semantics=("parallel",)),
    )(boxes, attention, f_weights)
